# baseline (speedup 1.0000x reference)
import functools

import jax
import jax.numpy as jnp
from jax import lax
from jax.experimental import pallas as pl
from jax.experimental.pallas import tpu as pltpu
from jax.experimental.pallas import tpu_sc as plsc

CHUNK = 128
NBUF = 8


@functools.lru_cache(maxsize=None)
def _make_gather(B, D):
    info = plsc.get_sparse_core_info()
    NC, NS = info.num_cores, info.num_subcores
    NW = NC * NS
    assert B % (NW * CHUNK * NBUF) == 0
    b_per_w = B // NW
    n_chunks = b_per_w // CHUNK
    n_groups = n_chunks // NBUF

    mesh = plsc.VectorSubcoreMesh(core_axis_name="c", subcore_axis_name="s")

    @functools.partial(
        pl.kernel,
        mesh=mesh,
        out_type=jax.ShapeDtypeStruct((B, D), jnp.float32),
        scratch_types=[
            pltpu.VMEM((n_chunks, CHUNK), jnp.int32),
            pltpu.VMEM((NBUF, CHUNK, D), jnp.float32),
            pltpu.SemaphoreType.DMA((NBUF,)),
            pltpu.SemaphoreType.DMA((NBUF,)),
        ],
        compiler_params=pltpu.CompilerParams(use_tc_tiling_on_sc=False),
    )
    def gather_kernel(idx_hbm, table_hbm, out_hbm, idx_v, rows_v, gsem, ssem):
        wid = lax.axis_index("s") * NC + lax.axis_index("c")
        base = wid * b_per_w
        pltpu.sync_copy(idx_hbm.at[wid], idx_v)

        def start_gather(j, b):
            pltpu.async_copy(
                table_hbm.at[pl.ds((base + j * CHUNK) % (1000000 - CHUNK), CHUNK)],
                rows_v.at[b],
                gsem.at[b],
            )

        def wait_gather(b):
            pltpu.make_async_copy(
                table_hbm.at[idx_v.at[0]], rows_v.at[b], gsem.at[b]
            ).wait()

        def start_scatter(j, b):
            pltpu.async_copy(
                rows_v.at[b], out_hbm.at[pl.ds(base + j * CHUNK, CHUNK)], ssem.at[b]
            )

        def wait_scatter(b):
            pltpu.make_async_copy(
                rows_v.at[b], out_hbm.at[pl.ds(base, CHUNK)], ssem.at[b]
            ).wait()

        for b in range(NBUF):
            start_gather(b, b)

        def group(g, carry):
            for b in range(NBUF):
                wait_gather(b)
                start_scatter(g * NBUF + b, b)

            @pl.when(g < n_groups - 1)
            def _prefetch():
                for b in range(NBUF):
                    wait_scatter(b)
                    start_gather((g + 1) * NBUF + b, b)

            return carry

        lax.fori_loop(0, n_groups, group, 0)
        for b in range(NBUF):
            wait_scatter(b)

    def run(idx, table):
        idx3 = idx.reshape(NW, n_chunks, CHUNK)
        return gather_kernel(idx3, table)

    return run


def kernel(token_ids, weight):
    B = token_ids.size
    idx = token_ids.reshape(B).astype(jnp.int32)
    out = _make_gather(B, weight.shape[1])(idx, weight)
    return out.reshape(*token_ids.shape, weight.shape[1])


# speedup vs baseline: 1.0001x; 1.0001x over previous
import functools

import jax
import jax.numpy as jnp
from jax import lax
from jax.experimental import pallas as pl
from jax.experimental.pallas import tpu as pltpu
from jax.experimental.pallas import tpu_sc as plsc

CHUNK = 256
NBUF = 4


@functools.lru_cache(maxsize=None)
def _make_gather(B, D):
    info = plsc.get_sparse_core_info()
    NC, NS = info.num_cores, info.num_subcores
    NW = NC * NS
    assert B % (NW * CHUNK * NBUF) == 0
    b_per_w = B // NW
    n_chunks = b_per_w // CHUNK
    n_groups = n_chunks // NBUF

    mesh = plsc.VectorSubcoreMesh(core_axis_name="c", subcore_axis_name="s")

    @functools.partial(
        pl.kernel,
        mesh=mesh,
        out_type=jax.ShapeDtypeStruct((B, D), jnp.float32),
        scratch_types=[
            pltpu.VMEM((n_chunks, CHUNK), jnp.int32),
            pltpu.VMEM((NBUF, CHUNK, D), jnp.float32),
            pltpu.SemaphoreType.DMA((NBUF,)),
            pltpu.SemaphoreType.DMA((NBUF,)),
        ],
        compiler_params=pltpu.CompilerParams(use_tc_tiling_on_sc=False),
    )
    def gather_kernel(idx_hbm, table_hbm, out_hbm, idx_v, rows_v, gsem, ssem):
        wid = lax.axis_index("s") * NC + lax.axis_index("c")
        base = wid * b_per_w
        pltpu.sync_copy(idx_hbm.at[wid], idx_v)

        def start_gather(j, b):
            pltpu.async_copy(table_hbm.at[idx_v.at[j]], rows_v.at[b], gsem.at[b])

        def wait_gather(b):
            pltpu.make_async_copy(
                table_hbm.at[idx_v.at[0]], rows_v.at[b], gsem.at[b]
            ).wait()

        def start_scatter(j, b):
            pltpu.async_copy(
                rows_v.at[b], out_hbm.at[pl.ds(base + j * CHUNK, CHUNK)], ssem.at[b]
            )

        def wait_scatter(b):
            pltpu.make_async_copy(
                rows_v.at[b], out_hbm.at[pl.ds(base, CHUNK)], ssem.at[b]
            ).wait()

        for b in range(NBUF):
            start_gather(b, b)

        def group(g, carry):
            for b in range(NBUF):
                wait_gather(b)
                start_scatter(g * NBUF + b, b)

            @pl.when(g < n_groups - 1)
            def _prefetch():
                for b in range(NBUF):
                    wait_scatter(b)
                    start_gather((g + 1) * NBUF + b, b)

            return carry

        lax.fori_loop(0, n_groups, group, 0)
        for b in range(NBUF):
            wait_scatter(b)

    def run(idx, table):
        idx3 = idx.reshape(NW, n_chunks, CHUNK)
        return gather_kernel(idx3, table)

    return run


def kernel(token_ids, weight):
    B = token_ids.size
    idx = token_ids.reshape(B).astype(jnp.int32)
    out = _make_gather(B, weight.shape[1])(idx, weight)
    return out.reshape(*token_ids.shape, weight.shape[1])


# speedup vs baseline: 1.0079x; 1.0078x over previous
import functools

import jax
import jax.numpy as jnp
from jax import lax
from jax.experimental import pallas as pl
from jax.experimental.pallas import tpu as pltpu
from jax.experimental.pallas import tpu_sc as plsc

CHUNK = 256
NBUF = 4
NBUF_S = 2


@functools.lru_cache(maxsize=None)
def _make_gather(B, D):
    info = plsc.get_sparse_core_info()
    NC, NS = info.num_cores, info.num_subcores
    NW = NC * NS
    assert B % (NW * CHUNK * NBUF) == 0
    b_per_w = B // NW
    n_chunks = b_per_w // CHUNK
    n_groups = n_chunks // NBUF

    mesh = plsc.VectorSubcoreMesh(core_axis_name="c", subcore_axis_name="s")

    @functools.partial(
        pl.kernel,
        mesh=mesh,
        out_type=jax.ShapeDtypeStruct((B, D), jnp.float32),
        scratch_types=[
            pltpu.VMEM((n_chunks, CHUNK), jnp.int32),
            pltpu.VMEM((NBUF, CHUNK, D), jnp.float32),
            pltpu.VMEM_SHARED((NS, NBUF_S, CHUNK, D), jnp.float32),
            pltpu.SemaphoreType.DMA((NBUF,)),
            pltpu.SemaphoreType.DMA((NBUF_S,)),
        ],
        compiler_params=pltpu.CompilerParams(use_tc_tiling_on_sc=False),
    )
    def gather_kernel(idx_hbm, table_hbm, out_hbm, idx_v, rows_v, sp, gsem, ssem):
        cid = lax.axis_index("c")
        sid = lax.axis_index("s")
        wid = sid * NC + cid
        base = wid * b_per_w
        pltpu.sync_copy(idx_hbm.at[wid], idx_v)

        def start_gather(j, b):
            pltpu.async_copy(table_hbm.at[idx_v.at[j]], rows_v.at[b], gsem.at[b])

        def wait_gather(b):
            pltpu.make_async_copy(
                table_hbm.at[idx_v.at[0]], rows_v.at[b], gsem.at[b]
            ).wait()

        def start_scatter(j, s):
            pltpu.async_copy(
                sp.at[sid, s], out_hbm.at[pl.ds(base + j * CHUNK, CHUNK)], ssem.at[s]
            )

        def wait_scatter(s):
            pltpu.make_async_copy(
                sp.at[sid, s], out_hbm.at[pl.ds(base, CHUNK)], ssem.at[s]
            ).wait()

        for b in range(NBUF):
            start_gather(b, b)

        def group(g, carry):
            for b in range(NBUF):
                s = b % NBUF_S
                wait_gather(b)
                if b >= NBUF_S:
                    wait_scatter(s)
                else:

                    @pl.when(g > 0)
                    def _drain():
                        wait_scatter(s)

                pltpu.sync_copy(rows_v.at[b], sp.at[sid, s])
                start_scatter(g * NBUF + b, s)

                @pl.when(g < n_groups - 1)
                def _prefetch():
                    start_gather((g + 1) * NBUF + b, b)

            return carry

        lax.fori_loop(0, n_groups, group, 0)
        for s in range(NBUF_S):
            wait_scatter(s)

    def run(idx, table):
        idx3 = idx.reshape(NW, n_chunks, CHUNK)
        return gather_kernel(idx3, table)

    return run


def kernel(token_ids, weight):
    B = token_ids.size
    idx = token_ids.reshape(B).astype(jnp.int32)
    out = _make_gather(B, weight.shape[1])(idx, weight)
    return out.reshape(*token_ids.shape, weight.shape[1])
